# baseline (speedup 1.0000x reference)
import functools

import jax
import jax.numpy as jnp
from jax import lax
from jax.experimental import pallas as pl
from jax.experimental.pallas import tpu as pltpu
from jax.experimental.pallas import tpu_sc as plsc

_CYCLE = 1000
_L = 200
_D = 64
_B = 4096
_EXT = _CYCLE + _L

_NC = 2
_NS = 16
_NW = _NC * _NS
_BPW = _B // _NW
_LANES = 16


@functools.partial(
    pl.kernel,
    out_type=jax.ShapeDtypeStruct((_L, _D, _B), jnp.float32),
    mesh=plsc.VectorSubcoreMesh(
        core_axis_name="c", subcore_axis_name="s",
        num_cores=_NC, num_subcores=_NS),
    compiler_params=pltpu.CompilerParams(
        use_tc_tiling_on_sc=False, needs_layout_passes=False),
    scratch_types=[
        pltpu.VMEM((_EXT * _D,), jnp.float32),
        pltpu.VMEM((_BPW,), jnp.int32),
        pltpu.VMEM((_LANES,), jnp.float32),
        pltpu.VMEM((_LANES,), jnp.int32),
        pltpu.VMEM((_D, _BPW), jnp.float32),
    ],
)
def _cycle_gather_t(phase_hbm, w_hbm, off_hbm, base_hbm, out_hbm,
                    table_v, phase_v, w_v, off_v, stage_v):
    wid = lax.axis_index("s") * _NC + lax.axis_index("c")
    bbase = wid * _BPW

    pltpu.sync_copy(base_hbm, table_v.at[pl.ds(0, _CYCLE * _D)])
    pltpu.sync_copy(base_hbm.at[pl.ds(0, _L * _D)],
                    table_v.at[pl.ds(_CYCLE * _D, _L * _D)])
    pltpu.sync_copy(phase_hbm.at[pl.ds(bbase, _BPW)], phase_v)
    pltpu.sync_copy(w_hbm, w_v)
    pltpu.sync_copy(off_hbm, off_v)

    w = w_v[...]
    off = off_v[...]
    curs = []
    for lg in range(_BPW // _LANES):
        ph = phase_v[pl.ds(lg * _LANES, _LANES)]
        shifted = (ph.astype(jnp.float32) * w).astype(jnp.int32) + off
        r = lax.rem(shifted, _CYCLE)
        r = jnp.where(r < 0, r + _CYCLE, r)
        curs.append(r * _D)

    def body(l, carry):
        curs = carry
        for d in range(_D):
            for lg in range(_BPW // _LANES):
                v = plsc.load_gather(table_v, [curs[lg] + d])
                stage_v[d, pl.ds(lg * _LANES, _LANES)] = v
        pltpu.sync_copy(stage_v, out_hbm.at[l, :, pl.ds(bbase, _BPW)])
        return tuple(c + _D for c in curs)

    lax.fori_loop(0, _L, body, tuple(curs))


def kernel(phase, length, base, w_ps):
    w16 = jnp.broadcast_to(jnp.reshape(w_ps, (1,)).astype(jnp.float32),
                           (_LANES,))
    off16 = jnp.broadcast_to(
        jnp.reshape(jnp.asarray(length, jnp.int32) - _L, (1,)), (_LANES,))
    y = _cycle_gather_t(phase, w16, off16, jnp.reshape(base, (-1,)))
    return jnp.transpose(y, (2, 0, 1))


# speedup vs baseline: 1.5042x; 1.5042x over previous
import functools

import jax
import jax.numpy as jnp
from jax import lax
from jax.experimental import pallas as pl
from jax.experimental.pallas import tpu as pltpu
from jax.experimental.pallas import tpu_sc as plsc

_CYCLE = 1000
_L = 200
_D = 64
_B = 4096
_EXT = _CYCLE + _L

_NC = 2
_NS = 16
_NW = _NC * _NS
_BPW = _B // _NW
_LANES = 16
_NLG = _BPW // _LANES
_NSEG = _D // _LANES
_SLAB = _D * _BPW


@functools.partial(
    pl.kernel,
    out_type=jax.ShapeDtypeStruct((_L, _D, _B), jnp.float32),
    mesh=plsc.VectorSubcoreMesh(
        core_axis_name="c", subcore_axis_name="s",
        num_cores=_NC, num_subcores=_NS),
    compiler_params=pltpu.CompilerParams(
        use_tc_tiling_on_sc=False, needs_layout_passes=False),
    scratch_types=[
        pltpu.VMEM((_EXT * _D,), jnp.float32),
        pltpu.VMEM((_BPW,), jnp.int32),
        pltpu.VMEM((_LANES,), jnp.float32),
        pltpu.VMEM((_LANES,), jnp.int32),
        pltpu.VMEM((_D, _BPW), jnp.float32),
        pltpu.VMEM((_D, _BPW), jnp.float32),
        pltpu.SemaphoreType.DMA,
        pltpu.SemaphoreType.DMA,
    ],
)
def _cycle_gather_t(phase_hbm, w_hbm, off_hbm, base_hbm, out_hbm,
                    table_v, phase_v, w_v, off_v, stage_a, stage_b,
                    sem0, sem1):
    wid = lax.axis_index("s") * _NC + lax.axis_index("c")
    bbase = wid * _BPW
    stages = (stage_a, stage_b)
    sems = (sem0, sem1)

    pltpu.sync_copy(base_hbm, table_v.at[pl.ds(0, _CYCLE * _D)])
    pltpu.sync_copy(base_hbm.at[pl.ds(0, _L * _D)],
                    table_v.at[pl.ds(_CYCLE * _D, _L * _D)])
    pltpu.sync_copy(phase_hbm.at[pl.ds(bbase, _BPW)], phase_v)
    pltpu.sync_copy(w_hbm, w_v)
    pltpu.sync_copy(off_hbm, off_v)

    w = w_v[...]
    off = off_v[...]
    curs = []
    for lg in range(_NLG):
        ph = phase_v[pl.ds(lg * _LANES, _LANES)]
        shifted = (ph.astype(jnp.float32) * w).astype(jnp.int32) + off
        r = lax.rem(shifted, _CYCLE)
        r = jnp.where(r < 0, r + _CYCLE, r)
        curs.append(r * _D)

    iota = lax.iota(jnp.int32, _LANES)
    rotv = [jnp.bitwise_and(iota + r, _LANES - 1) for r in range(_LANES)]
    bvs = [iota + lg * _LANES for lg in range(_NLG)]

    def body(lp, carry):
        curs = carry
        for half in range(2):
            stg = stages[half]
            sem = sems[half]

            @pl.when(lp > 0)
            def _wait_prev():
                pltpu.make_async_copy(
                    stg, out_hbm.at[0, :, pl.ds(bbase, _BPW)], sem).wait()

            for seg in range(_NSEG):
                dvs = [rotv[r] + seg * _LANES for r in range(_LANES)]
                for lg in range(_NLG):
                    curls = curs[lg] + (half * _D + seg * _LANES)
                    for r in range(_LANES):
                        v = plsc.load_gather(table_v, [curls + rotv[r]])
                        plsc.store_scatter(stg, [dvs[r], bvs[lg]], v)
            pltpu.async_copy(
                stg, out_hbm.at[2 * lp + half, :, pl.ds(bbase, _BPW)], sem)
        return tuple(c + 2 * _D for c in curs)

    lax.fori_loop(0, _L // 2, body, tuple(curs))
    for half in range(2):
        pltpu.make_async_copy(
            stages[half],
            out_hbm.at[0, :, pl.ds(bbase, _BPW)], sems[half]).wait()


def kernel(phase, length, base, w_ps):
    w16 = jnp.broadcast_to(jnp.reshape(w_ps, (1,)).astype(jnp.float32),
                           (_LANES,))
    off16 = jnp.broadcast_to(
        jnp.reshape(jnp.asarray(length, jnp.int32) - _L, (1,)), (_LANES,))
    y = _cycle_gather_t(phase, w16, off16, jnp.reshape(base, (-1,)))
    return jnp.transpose(y, (2, 0, 1))


# speedup vs baseline: 1.8569x; 1.2345x over previous
import functools

import jax
import jax.numpy as jnp
from jax import lax
from jax.experimental import pallas as pl
from jax.experimental.pallas import tpu as pltpu
from jax.experimental.pallas import tpu_sc as plsc

_CYCLE = 1000
_L = 200
_D = 64
_B = 4096
_EXT = _CYCLE + _L

_NC = 2
_NS = 16
_NW = _NC * _NS
_BPW = _B // _NW
_LANES = 16
_NLG = _BPW // _LANES
_NSEG = _D // _LANES
_SLAB = _D * _BPW


@functools.partial(
    pl.kernel,
    out_type=jax.ShapeDtypeStruct((_L, _D, _B), jnp.float32),
    mesh=plsc.VectorSubcoreMesh(
        core_axis_name="c", subcore_axis_name="s",
        num_cores=_NC, num_subcores=_NS),
    compiler_params=pltpu.CompilerParams(
        use_tc_tiling_on_sc=False, needs_layout_passes=False),
    scratch_types=[
        pltpu.VMEM((_EXT * _D,), jnp.float32),
        pltpu.VMEM((_BPW,), jnp.int32),
        pltpu.VMEM((_LANES,), jnp.float32),
        pltpu.VMEM((_LANES,), jnp.int32),
        pltpu.VMEM((_D, _BPW), jnp.float32),
        pltpu.VMEM((_D, _BPW), jnp.float32),
        pltpu.VMEM((_LANES, _LANES), jnp.int32),
        pltpu.SemaphoreType.DMA,
        pltpu.SemaphoreType.DMA,
    ],
)
def _cycle_gather_t(phase_hbm, w_hbm, off_hbm, base_hbm, out_hbm,
                    table_v, phase_v, w_v, off_v, stage_a, stage_b,
                    rotm_v, sem0, sem1):
    wid = lax.axis_index("s") * _NC + lax.axis_index("c")
    bbase = wid * _BPW
    stages = (stage_a, stage_b)
    sems = (sem0, sem1)

    pltpu.sync_copy(base_hbm, table_v.at[pl.ds(0, _CYCLE * _D)])
    pltpu.sync_copy(base_hbm.at[pl.ds(0, _L * _D)],
                    table_v.at[pl.ds(_CYCLE * _D, _L * _D)])
    pltpu.sync_copy(phase_hbm.at[pl.ds(bbase, _BPW)], phase_v)
    pltpu.sync_copy(w_hbm, w_v)
    pltpu.sync_copy(off_hbm, off_v)

    w = w_v[...]
    off = off_v[...]
    curs = []
    for lg in range(_NLG):
        ph = phase_v[pl.ds(lg * _LANES, _LANES)]
        shifted = (ph.astype(jnp.float32) * w).astype(jnp.int32) + off
        r = lax.rem(shifted, _CYCLE)
        r = jnp.where(r < 0, r + _CYCLE, r)
        curs.append(r * _D)

    iota = lax.iota(jnp.int32, _LANES)
    for r in range(_LANES):
        rotm_v[r, :] = jnp.bitwise_and(iota + r, _LANES - 1)

    def body(lp, carry):
        curs = carry
        for half in range(2):
            stg = stages[half]
            sem = sems[half]

            @pl.when(lp > 0)
            def _wait_prev():
                pltpu.make_async_copy(
                    stg, out_hbm.at[0, :, pl.ds(bbase, _BPW)], sem).wait()

            for r in range(_LANES):
                rv = rotm_v[r, :]
                for lg in range(_NLG):
                    gbase = curs[lg] + rv
                    bv = iota + lg * _LANES
                    for seg in range(_NSEG):
                        v = plsc.load_gather(
                            table_v,
                            [gbase + (half * _D + seg * _LANES)])
                        plsc.store_scatter(
                            stg, [rv + seg * _LANES, bv], v)
            pltpu.async_copy(
                stg, out_hbm.at[2 * lp + half, :, pl.ds(bbase, _BPW)], sem)
        return tuple(c + 2 * _D for c in curs)

    lax.fori_loop(0, _L // 2, body, tuple(curs))
    for half in range(2):
        pltpu.make_async_copy(
            stages[half],
            out_hbm.at[0, :, pl.ds(bbase, _BPW)], sems[half]).wait()


def kernel(phase, length, base, w_ps):
    w16 = jnp.broadcast_to(jnp.reshape(w_ps, (1,)).astype(jnp.float32),
                           (_LANES,))
    off16 = jnp.broadcast_to(
        jnp.reshape(jnp.asarray(length, jnp.int32) - _L, (1,)), (_LANES,))
    y = _cycle_gather_t(phase, w16, off16, jnp.reshape(base, (-1,)))
    return jnp.transpose(y, (2, 0, 1))


# speedup vs baseline: 3.4901x; 1.8795x over previous
import functools

import jax
import jax.numpy as jnp
from jax import lax
from jax.experimental import pallas as pl
from jax.experimental.pallas import tpu as pltpu
from jax.experimental.pallas import tpu_sc as plsc

_CYCLE = 1000
_L = 200
_D = 64
_B = 4096
_EXT = _CYCLE + _L - 1

_NC = 2
_NS = 16
_NW = _NC * _NS
_BPW = _B // _NW
_LANES = 16


@functools.partial(
    pl.kernel,
    out_type=jax.ShapeDtypeStruct((_B, _L * _D), jnp.float32),
    mesh=plsc.VectorSubcoreMesh(
        core_axis_name="c", subcore_axis_name="s",
        num_cores=_NC, num_subcores=_NS),
    compiler_params=pltpu.CompilerParams(use_tc_tiling_on_sc=False),
    scratch_types=[
        pltpu.VMEM((_EXT * _D,), jnp.float32),
        pltpu.VMEM((_BPW,), jnp.int32),
        pltpu.VMEM((_LANES,), jnp.float32),
        pltpu.VMEM((_LANES,), jnp.int32),
        pltpu.VMEM((_BPW,), jnp.int32),
        pltpu.SemaphoreType.DMA,
    ],
)
def _cycle_gather(phase_hbm, w_hbm, off_hbm, base_hbm, out_hbm,
                  table_v, phase_v, w_v, off_v, starts_v, sem):
    wid = lax.axis_index("s") * _NC + lax.axis_index("c")
    bbase = wid * _BPW

    pltpu.sync_copy(base_hbm, table_v.at[pl.ds(0, _CYCLE * _D)])
    pltpu.sync_copy(base_hbm.at[pl.ds(0, (_L - 1) * _D)],
                    table_v.at[pl.ds(_CYCLE * _D, (_L - 1) * _D)])
    pltpu.sync_copy(phase_hbm.at[pl.ds(bbase, _BPW)], phase_v)
    pltpu.sync_copy(w_hbm, w_v)
    pltpu.sync_copy(off_hbm, off_v)

    w = w_v[...]
    off = off_v[...]
    for i in range(_BPW // _LANES):
        ph = phase_v[pl.ds(i * _LANES, _LANES)]
        shifted = (ph.astype(jnp.float32) * w).astype(jnp.int32) + off
        r = lax.rem(shifted, _CYCLE)
        r = jnp.where(r < 0, r + _CYCLE, r)
        starts_v[pl.ds(i * _LANES, _LANES)] = r

    window = 32
    pending = []
    for g in range(_BPW // _LANES):
        sv = starts_v[pl.ds(g * _LANES, _LANES)]
        for j in range(_LANES):
            s = sv[j]
            c = pltpu.async_copy(table_v.at[pl.ds(s * _D, _L * _D)],
                                 out_hbm.at[bbase + g * _LANES + j], sem)
            pending.append(c)
            if len(pending) > window:
                pending.pop(0).wait()
    for c in pending:
        c.wait()


def kernel(phase, length, base, w_ps):
    w16 = jnp.broadcast_to(jnp.reshape(w_ps, (1,)).astype(jnp.float32),
                           (_LANES,))
    off16 = jnp.broadcast_to(
        jnp.reshape(jnp.asarray(length, jnp.int32) - _L, (1,)), (_LANES,))
    flat = _cycle_gather(phase, w16, off16, jnp.reshape(base, (-1,)))
    return jnp.reshape(flat, (_B, _L, _D))
